# baseline (speedup 1.0000x reference)
import jax
import jax.numpy as jnp
from jax import lax
from jax.experimental import pallas as pl
from jax.experimental.pallas import tpu as pltpu

_B, _P, _N, _S = 8, 20000, 128, 8
_PCL = 2048
_PP = 20480
_NCHUNK = _PP // _PCL
_G = _PP // 128
_EPS = 1e-6
_HI = lax.Precision.HIGHEST


def _softplus_pos(x):
    return jnp.maximum(x, 0.0) + jnp.log1p(jnp.exp(-jnp.abs(x)))


def _loss_kernel(ch_ref, bg_ref, sg_ref, cg_ref, zg_ref, tb_ref, tsc_ref,
                 tctx_ref, out_ref):
    f32 = jnp.float32
    tb = tb_ref[0]
    tx0 = tb[:, 0:1]
    ty0 = tb[:, 1:2]
    tx1 = tb[:, 2:3]
    ty1 = tb[:, 3:4]
    area_b = (tx1 - tx0) * (ty1 - ty0)

    best = jnp.full((_N, 1), -jnp.inf, f32)
    bidx = jnp.zeros((_N, 1), jnp.int32)

    for i in range(_NCHUNK):
        off = i * _PCL
        x0 = ch_ref[0, 0:1, off:off + _PCL]
        y0 = ch_ref[0, 1:2, off:off + _PCL]
        x1 = ch_ref[0, 2:3, off:off + _PCL]
        y1 = ch_ref[0, 3:4, off:off + _PCL]
        iw = jnp.maximum(jnp.minimum(x1, tx1) - jnp.maximum(x0, tx0), 0.0)
        ih = jnp.maximum(jnp.minimum(y1, ty1) - jnp.maximum(y0, ty0), 0.0)
        inter = iw * ih
        area_a = (x1 - x0) * (y1 - y0)
        union = area_a + area_b - inter
        iou = inter / jnp.maximum(union, 1e-9)
        cmax = jnp.max(iou, axis=1, keepdims=True)
        lid = lax.broadcasted_iota(jnp.int32, (_N, _PCL), 1) + off
        cand = jnp.where(iou == cmax, lid, jnp.int32(2**31 - 1))
        cidx = jnp.min(cand, axis=1, keepdims=True)
        better = cmax > best
        best = jnp.where(better, cmax, best)
        bidx = jnp.where(better, cidx, bidx)

    bg = bg_ref[0]
    sg = sg_ref[0]
    cg = cg_ref[0]
    zg = zg_ref[0]
    fb_box = jnp.sum(bg * bg)
    fb_scales = jnp.sum(sg * sg)
    fb_ctx = jnp.sum(cg * cg)
    fb_sc = jnp.sum(zg * zg)
    dsum = jnp.sum(_softplus_pos(zg)) - (_PP - _P) * jnp.log1p(f32(1.0))

    vm = (best > 0.5).astype(f32)
    cnt = jnp.sum(vm)
    denom = jnp.maximum(cnt, 1.0)

    grp = bidx // 128
    lane = jnp.remainder(bidx, 128)
    ohgrp = (lax.broadcasted_iota(jnp.int32, (_N, _G), 1) == grp).astype(f32)
    gb = lax.dot_general(ohgrp, bg, (((1,), (0,)), ((), ())),
                         precision=_HI, preferred_element_type=f32)
    gs = lax.dot_general(ohgrp, sg, (((1,), (0,)), ((), ())),
                         precision=_HI, preferred_element_type=f32)
    gc = lax.dot_general(ohgrp, cg, (((1,), (0,)), ((), ())),
                         precision=_HI, preferred_element_type=f32)
    gz = lax.dot_general(ohgrp, zg, (((1,), (0,)), ((), ())),
                         precision=_HI, preferred_element_type=f32)

    li512 = lax.broadcasted_iota(jnp.int32, (_N, 512), 1)
    g4c = []
    for c in range(4):
        m = (li512 == lane * 4 + c).astype(f32)
        g4c.append(jnp.sum(gb * m, axis=1, keepdims=True))
    g4 = jnp.concatenate(g4c, axis=1)
    li1024 = lax.broadcasted_iota(jnp.int32, (_N, 1024), 1)
    g8c = []
    for c in range(_S):
        m = (li1024 == lane * 8 + c).astype(f32)
        g8c.append(jnp.sum(gs * m, axis=1, keepdims=True))
    g8 = jnp.concatenate(g8c, axis=1)
    ohlane = (lax.broadcasted_iota(jnp.int32, (_N, 128), 1) == lane).astype(f32)
    g1 = jnp.sum(gc * ohlane, axis=1, keepdims=True)
    svals = jnp.sum(gz * ohlane, axis=1, keepdims=True)

    bidx_f = bidx.astype(f32)
    eye = (lax.broadcasted_iota(jnp.int32, (_N, _N), 0)
           == lax.broadcasted_iota(jnp.int32, (_N, _N), 1)).astype(f32)
    bv = jnp.concatenate([bidx_f, vm], axis=1)
    bvt = lax.dot_general(bv, eye, (((0,), (0,)), ((), ())),
                          precision=_HI, preferred_element_type=f32)
    bidx_t = bvt[0:1, :]
    vm_t = bvt[1:2, :]
    tri = (lax.broadcasted_iota(jnp.int32, (_N, _N), 0)
           > lax.broadcasted_iota(jnp.int32, (_N, _N), 1)).astype(f32)
    dup = (bidx_f == bidx_t).astype(f32) * vm * vm_t * tri
    dupped = jnp.max(dup, axis=1, keepdims=True)
    msum = jnp.sum(vm * (1.0 - dupped) * svals)

    d = g4 - tb
    ad = jnp.abs(d)
    sl1 = jnp.where(ad < 1.0, 0.5 * d * d, ad - 0.5)
    box_m = jnp.sum(sl1 * vm) / (denom * 4.0)

    m8 = jnp.max(g8, axis=1, keepdims=True)
    lse = jnp.log(jnp.sum(jnp.exp(g8 - m8), axis=1, keepdims=True)) + m8
    labels = tsc_ref[0]
    scol = lax.broadcasted_iota(jnp.int32, (_N, _S), 1)
    onehot = (scol == labels).astype(f32)
    picked = jnp.sum(g8 * onehot, axis=1, keepdims=True)
    sc_m = jnp.sum((lse - picked) * vm) / denom

    tc = tctx_ref[0]
    bce = _softplus_pos(g1) - g1 * tc
    ctx_m = jnp.sum(bce * vm) / denom

    conf_m = (dsum - msum) / _P

    any_v = cnt > 0.0
    box_o = jnp.where(any_v, box_m, fb_box / (_P * 4.0) * _EPS)
    scale_o = jnp.where(any_v, sc_m, fb_scales / (_P * _S) * _EPS)
    ctx_o = jnp.where(any_v, ctx_m, fb_ctx / _P * _EPS)
    conf_o = jnp.where(any_v, conf_m, fb_sc / _P * _EPS)

    li = lax.broadcasted_iota(jnp.int32, (1, 128), 1)
    vals = (jnp.where(li == 0, box_o, 0.0) + jnp.where(li == 1, scale_o, 0.0)
            + jnp.where(li == 2, ctx_o, 0.0) + jnp.where(li == 3, conf_o, 0.0))
    out_ref[0] = vals


def kernel(scores, boxes, scales, context_scores, target_boxes, target_scales,
           target_context, confidence):
    del confidence
    pad = _PP - _P
    boxes_p = jnp.pad(boxes, ((0, 0), (0, pad), (0, 0)))
    ch = jnp.transpose(boxes_p, (0, 2, 1))
    bg = boxes_p.reshape(_B, _G, 512)
    sg = jnp.pad(scales, ((0, 0), (0, pad), (0, 0))).reshape(_B, _G, 1024)
    cg = jnp.pad(context_scores, ((0, 0), (0, pad))).reshape(_B, _G, 128)
    zg = jnp.pad(scores, ((0, 0), (0, pad))).reshape(_B, _G, 128)
    tsc = target_scales.astype(jnp.int32)[:, :, None]
    tctx = target_context[:, :, None]
    out = pl.pallas_call(
        _loss_kernel,
        grid=(_B,),
        in_specs=[
            pl.BlockSpec((1, 4, _PP), lambda b: (b, 0, 0)),
            pl.BlockSpec((1, _G, 512), lambda b: (b, 0, 0)),
            pl.BlockSpec((1, _G, 1024), lambda b: (b, 0, 0)),
            pl.BlockSpec((1, _G, 128), lambda b: (b, 0, 0)),
            pl.BlockSpec((1, _G, 128), lambda b: (b, 0, 0)),
            pl.BlockSpec((1, _N, 4), lambda b: (b, 0, 0)),
            pl.BlockSpec((1, _N, 1), lambda b: (b, 0, 0)),
            pl.BlockSpec((1, _N, 1), lambda b: (b, 0, 0)),
        ],
        out_specs=pl.BlockSpec((1, 1, 128), lambda b: (b, 0, 0)),
        out_shape=jax.ShapeDtypeStruct((_B, 1, 128), jnp.float32),
        compiler_params=pltpu.CompilerParams(
            dimension_semantics=("parallel",)),
    )(ch, bg, sg, cg, zg, target_boxes, tsc, tctx)
    res = out[:, 0, :]
    box_loss = jnp.mean(res[:, 0])
    scale_loss = jnp.mean(res[:, 1])
    context_loss = jnp.mean(res[:, 2])
    conf_loss = jnp.mean(res[:, 3])
    total = box_loss + scale_loss + context_loss + conf_loss
    return {"loss": total, "box_loss": box_loss, "scale_loss": scale_loss,
            "context_loss": context_loss, "conf_loss": conf_loss}


# speedup vs baseline: 2.7128x; 2.7128x over previous
import jax
import jax.numpy as jnp
from jax import lax
from jax.experimental import pallas as pl
from jax.experimental.pallas import tpu as pltpu

_B, _P, _N, _S = 8, 20000, 128, 8
_PCL = 2048
_PP = 20480
_NCHUNK = _PP // _PCL
_C = 14
_G = _PP // 128
_EPS = 1e-6
_HI = lax.Precision.HIGHEST


def _softplus_pos(x):
    return jnp.maximum(x, 0.0) + jnp.log1p(jnp.exp(-jnp.abs(x)))


def _loss_kernel(ch_ref, tb_ref, tsc_ref, tctx_ref, out_ref, scr_ref):
    f32 = jnp.float32
    tb = tb_ref[0]
    tx0 = tb[:, 0:1]
    ty0 = tb[:, 1:2]
    tx1 = tb[:, 2:3]
    ty1 = tb[:, 3:4]
    area_b = (tx1 - tx0) * (ty1 - ty0)

    for g in range(_G):
        scr_ref[:, g, :] = ch_ref[0, :, g * 128:(g + 1) * 128]

    best = jnp.full((_N, 1), -jnp.inf, f32)
    bidx = jnp.zeros((_N, 1), jnp.int32)

    for i in range(_NCHUNK):
        off = i * _PCL
        x0 = ch_ref[0, 0:1, off:off + _PCL]
        y0 = ch_ref[0, 1:2, off:off + _PCL]
        x1 = ch_ref[0, 2:3, off:off + _PCL]
        y1 = ch_ref[0, 3:4, off:off + _PCL]
        iw = jnp.maximum(jnp.minimum(x1, tx1) - jnp.maximum(x0, tx0), 0.0)
        ih = jnp.maximum(jnp.minimum(y1, ty1) - jnp.maximum(y0, ty0), 0.0)
        inter = iw * ih
        area_a = (x1 - x0) * (y1 - y0)
        union = area_a + area_b - inter
        iou = inter / jnp.maximum(union, 1e-9)
        cmax = jnp.max(iou, axis=1, keepdims=True)
        lid = lax.broadcasted_iota(jnp.int32, (_N, _PCL), 1) + off
        cand = jnp.where(iou == cmax, lid, jnp.int32(2**31 - 1))
        cidx = jnp.min(cand, axis=1, keepdims=True)
        better = cmax > best
        best = jnp.where(better, cmax, best)
        bidx = jnp.where(better, cidx, bidx)

    fb_box = f32(0.0)
    fb_scales = f32(0.0)
    for c in range(4):
        row = ch_ref[0, c:c + 1, :]
        fb_box = fb_box + jnp.sum(row * row)
    for c in range(4, 12):
        row = ch_ref[0, c:c + 1, :]
        fb_scales = fb_scales + jnp.sum(row * row)
    crow = ch_ref[0, 12:13, :]
    fb_ctx = jnp.sum(crow * crow)
    srow = ch_ref[0, 13:14, :]
    fb_sc = jnp.sum(srow * srow)
    dsum = jnp.sum(_softplus_pos(srow)) - (_PP - _P) * jnp.log1p(f32(1.0))

    vm = (best > 0.5).astype(f32)
    cnt = jnp.sum(vm)
    denom = jnp.maximum(cnt, 1.0)

    grp = bidx // 128
    lane = jnp.remainder(bidx, 128)
    ohgrp = (lax.broadcasted_iota(jnp.int32, (_N, _G), 1) == grp).astype(f32)
    ohlane = (lax.broadcasted_iota(jnp.int32, (_N, 128), 1) == lane).astype(f32)
    gch = []
    for c in range(_C):
        part = lax.dot_general(ohgrp, scr_ref[c], (((1,), (0,)), ((), ())),
                               precision=_HI, preferred_element_type=f32)
        gch.append(jnp.sum(part * ohlane, axis=1, keepdims=True))
    g4 = jnp.concatenate(gch[0:4], axis=1)
    g8 = jnp.concatenate(gch[4:12], axis=1)
    g1 = gch[12]
    svals = gch[13]

    bidx_f = bidx.astype(f32)
    eye = (lax.broadcasted_iota(jnp.int32, (_N, _N), 0)
           == lax.broadcasted_iota(jnp.int32, (_N, _N), 1)).astype(f32)
    bv = jnp.concatenate([bidx_f, vm], axis=1)
    bvt = lax.dot_general(bv, eye, (((0,), (0,)), ((), ())),
                          precision=_HI, preferred_element_type=f32)
    bidx_t = bvt[0:1, :]
    vm_t = bvt[1:2, :]
    tri = (lax.broadcasted_iota(jnp.int32, (_N, _N), 0)
           > lax.broadcasted_iota(jnp.int32, (_N, _N), 1)).astype(f32)
    dup = (bidx_f == bidx_t).astype(f32) * vm * vm_t * tri
    dupped = jnp.max(dup, axis=1, keepdims=True)
    msum = jnp.sum(vm * (1.0 - dupped) * svals)

    d = g4 - tb
    ad = jnp.abs(d)
    sl1 = jnp.where(ad < 1.0, 0.5 * d * d, ad - 0.5)
    box_m = jnp.sum(sl1 * vm) / (denom * 4.0)

    m8 = jnp.max(g8, axis=1, keepdims=True)
    lse = jnp.log(jnp.sum(jnp.exp(g8 - m8), axis=1, keepdims=True)) + m8
    labels = tsc_ref[0]
    scol = lax.broadcasted_iota(jnp.int32, (_N, _S), 1)
    onehot = (scol == labels).astype(f32)
    picked = jnp.sum(g8 * onehot, axis=1, keepdims=True)
    sc_m = jnp.sum((lse - picked) * vm) / denom

    tc = tctx_ref[0]
    bce = _softplus_pos(g1) - g1 * tc
    ctx_m = jnp.sum(bce * vm) / denom

    conf_m = (dsum - msum) / _P

    any_v = cnt > 0.0
    box_o = jnp.where(any_v, box_m, fb_box / (_P * 4.0) * _EPS)
    scale_o = jnp.where(any_v, sc_m, fb_scales / (_P * _S) * _EPS)
    ctx_o = jnp.where(any_v, ctx_m, fb_ctx / _P * _EPS)
    conf_o = jnp.where(any_v, conf_m, fb_sc / _P * _EPS)

    li = lax.broadcasted_iota(jnp.int32, (1, 128), 1)
    vals = (jnp.where(li == 0, box_o, 0.0) + jnp.where(li == 1, scale_o, 0.0)
            + jnp.where(li == 2, ctx_o, 0.0) + jnp.where(li == 3, conf_o, 0.0))
    out_ref[0] = vals


def kernel(scores, boxes, scales, context_scores, target_boxes, target_scales,
           target_context, confidence):
    del confidence
    pad = _PP - _P
    ch = jnp.concatenate([
        jnp.transpose(jnp.pad(boxes, ((0, 0), (0, pad), (0, 0))), (0, 2, 1)),
        jnp.transpose(jnp.pad(scales, ((0, 0), (0, pad), (0, 0))), (0, 2, 1)),
        jnp.pad(context_scores, ((0, 0), (0, pad)))[:, None, :],
        jnp.pad(scores, ((0, 0), (0, pad)))[:, None, :],
    ], axis=1)
    tsc = target_scales.astype(jnp.int32)[:, :, None]
    tctx = target_context[:, :, None]
    out = pl.pallas_call(
        _loss_kernel,
        grid=(_B,),
        in_specs=[
            pl.BlockSpec((1, _C, _PP), lambda b: (b, 0, 0)),
            pl.BlockSpec((1, _N, 4), lambda b: (b, 0, 0)),
            pl.BlockSpec((1, _N, 1), lambda b: (b, 0, 0)),
            pl.BlockSpec((1, _N, 1), lambda b: (b, 0, 0)),
        ],
        out_specs=pl.BlockSpec((1, 1, 128), lambda b: (b, 0, 0)),
        out_shape=jax.ShapeDtypeStruct((_B, 1, 128), jnp.float32),
        scratch_shapes=[pltpu.VMEM((_C, _G, 128), jnp.float32)],
        compiler_params=pltpu.CompilerParams(
            dimension_semantics=("parallel",)),
    )(ch, target_boxes, tsc, tctx)
    res = out[:, 0, :]
    box_loss = jnp.mean(res[:, 0])
    scale_loss = jnp.mean(res[:, 1])
    context_loss = jnp.mean(res[:, 2])
    conf_loss = jnp.mean(res[:, 3])
    total = box_loss + scale_loss + context_loss + conf_loss
    return {"loss": total, "box_loss": box_loss, "scale_loss": scale_loss,
            "context_loss": context_loss, "conf_loss": conf_loss}


# speedup vs baseline: 3.0348x; 1.1187x over previous
import jax
import jax.numpy as jnp
from jax import lax
from jax.experimental import pallas as pl
from jax.experimental.pallas import tpu as pltpu

_B, _P, _N, _S = 8, 20000, 128, 8
_PCL = 2048
_PP = 20480
_NCHUNK = _PP // _PCL
_C = 14
_G = _PP // 128
_EPS = 1e-6
_HI = lax.Precision.HIGHEST


def _softplus_pos(x):
    return jnp.maximum(x, 0.0) + jnp.log1p(jnp.exp(-jnp.abs(x)))


def _loss_kernel(ch_ref, chg_ref, tb_ref, tsc_ref, tctx_ref, out_ref, scr_ref):
    f32 = jnp.float32
    tb = tb_ref[0]
    tx0 = tb[:, 0:1]
    ty0 = tb[:, 1:2]
    tx1 = tb[:, 2:3]
    ty1 = tb[:, 3:4]
    area_b = (tx1 - tx0) * (ty1 - ty0)

    scr_ref[0] = jnp.broadcast_to(tx0, (_N, _PCL))
    scr_ref[1] = jnp.broadcast_to(ty0, (_N, _PCL))
    scr_ref[2] = jnp.broadcast_to(tx1, (_N, _PCL))
    scr_ref[3] = jnp.broadcast_to(ty1, (_N, _PCL))
    scr_ref[4] = jnp.broadcast_to(area_b, (_N, _PCL))
    scr_ref[5] = lax.broadcasted_iota(jnp.int32, (_N, _PCL), 1).astype(f32)

    best = jnp.full((_N, 1), -jnp.inf, f32)
    bidx = jnp.zeros((_N, 1), jnp.int32)

    for i in range(_NCHUNK):
        off = i * _PCL
        x0 = ch_ref[0, 0:1, off:off + _PCL]
        y0 = ch_ref[0, 1:2, off:off + _PCL]
        x1 = ch_ref[0, 2:3, off:off + _PCL]
        y1 = ch_ref[0, 3:4, off:off + _PCL]
        t0 = scr_ref[0]
        t1 = scr_ref[1]
        t2 = scr_ref[2]
        t3 = scr_ref[3]
        ab = scr_ref[4]
        lid0 = scr_ref[5]
        iw = jnp.maximum(jnp.minimum(x1, t2) - jnp.maximum(x0, t0), 0.0)
        ih = jnp.maximum(jnp.minimum(y1, t3) - jnp.maximum(y0, t1), 0.0)
        inter = iw * ih
        area_a = (x1 - x0) * (y1 - y0)
        union = area_a + ab - inter
        iou = inter / jnp.maximum(union, 1e-9)
        cmax = jnp.max(iou, axis=1, keepdims=True)
        cand = jnp.where(iou == cmax, lid0, jnp.inf)
        cidx = (jnp.min(cand, axis=1, keepdims=True).astype(jnp.int32)
                + off)
        better = cmax > best
        best = jnp.where(better, cmax, best)
        bidx = jnp.where(better, cidx, bidx)

    fb_box = f32(0.0)
    fb_scales = f32(0.0)
    for c in range(4):
        blk = chg_ref[0, :, c * 128:(c + 1) * 128]
        fb_box = fb_box + jnp.sum(blk * blk)
    for c in range(4, 12):
        blk = chg_ref[0, :, c * 128:(c + 1) * 128]
        fb_scales = fb_scales + jnp.sum(blk * blk)
    cblk = chg_ref[0, :, 12 * 128:13 * 128]
    fb_ctx = jnp.sum(cblk * cblk)
    sblk = chg_ref[0, :, 13 * 128:14 * 128]
    fb_sc = jnp.sum(sblk * sblk)
    dsum = jnp.sum(_softplus_pos(sblk)) - (_PP - _P) * jnp.log1p(f32(1.0))

    vm = (best > 0.5).astype(f32)
    cnt = jnp.sum(vm)
    denom = jnp.maximum(cnt, 1.0)

    grp = bidx // 128
    lane = jnp.remainder(bidx, 128)
    ohgrp = (lax.broadcasted_iota(jnp.int32, (_N, _G), 1) == grp).astype(f32)
    ohlane = (lax.broadcasted_iota(jnp.int32, (_N, 128), 1) == lane).astype(f32)
    gall = lax.dot_general(ohgrp, chg_ref[0], (((1,), (0,)), ((), ())),
                           precision=_HI, preferred_element_type=f32)
    gch = []
    for c in range(_C):
        part = gall[:, c * 128:(c + 1) * 128]
        gch.append(jnp.sum(part * ohlane, axis=1, keepdims=True))
    g4 = jnp.concatenate(gch[0:4], axis=1)
    g8 = jnp.concatenate(gch[4:12], axis=1)
    g1 = gch[12]
    svals = gch[13]

    bidx_f = bidx.astype(f32)
    eye = (lax.broadcasted_iota(jnp.int32, (_N, _N), 0)
           == lax.broadcasted_iota(jnp.int32, (_N, _N), 1)).astype(f32)
    bv = jnp.concatenate([bidx_f, vm], axis=1)
    bvt = lax.dot_general(bv, eye, (((0,), (0,)), ((), ())),
                          precision=_HI, preferred_element_type=f32)
    bidx_t = bvt[0:1, :]
    vm_t = bvt[1:2, :]
    tri = (lax.broadcasted_iota(jnp.int32, (_N, _N), 0)
           > lax.broadcasted_iota(jnp.int32, (_N, _N), 1)).astype(f32)
    dup = (bidx_f == bidx_t).astype(f32) * vm * vm_t * tri
    dupped = jnp.max(dup, axis=1, keepdims=True)
    msum = jnp.sum(vm * (1.0 - dupped) * svals)

    d = g4 - tb
    ad = jnp.abs(d)
    sl1 = jnp.where(ad < 1.0, 0.5 * d * d, ad - 0.5)
    box_m = jnp.sum(sl1 * vm) / (denom * 4.0)

    m8 = jnp.max(g8, axis=1, keepdims=True)
    lse = jnp.log(jnp.sum(jnp.exp(g8 - m8), axis=1, keepdims=True)) + m8
    labels = tsc_ref[0]
    scol = lax.broadcasted_iota(jnp.int32, (_N, _S), 1)
    onehot = (scol == labels).astype(f32)
    picked = jnp.sum(g8 * onehot, axis=1, keepdims=True)
    sc_m = jnp.sum((lse - picked) * vm) / denom

    tc = tctx_ref[0]
    bce = _softplus_pos(g1) - g1 * tc
    ctx_m = jnp.sum(bce * vm) / denom

    conf_m = (dsum - msum) / _P

    any_v = cnt > 0.0
    box_o = jnp.where(any_v, box_m, fb_box / (_P * 4.0) * _EPS)
    scale_o = jnp.where(any_v, sc_m, fb_scales / (_P * _S) * _EPS)
    ctx_o = jnp.where(any_v, ctx_m, fb_ctx / _P * _EPS)
    conf_o = jnp.where(any_v, conf_m, fb_sc / _P * _EPS)

    li = lax.broadcasted_iota(jnp.int32, (1, 128), 1)
    vals = (jnp.where(li == 0, box_o, 0.0) + jnp.where(li == 1, scale_o, 0.0)
            + jnp.where(li == 2, ctx_o, 0.0) + jnp.where(li == 3, conf_o, 0.0))
    out_ref[0] = vals


def kernel(scores, boxes, scales, context_scores, target_boxes, target_scales,
           target_context, confidence):
    del confidence
    pad = _PP - _P
    boxes_p = jnp.pad(boxes, ((0, 0), (0, pad), (0, 0)))
    scales_p = jnp.pad(scales, ((0, 0), (0, pad), (0, 0)))
    ctx_p = jnp.pad(context_scores, ((0, 0), (0, pad)))
    scores_p = jnp.pad(scores, ((0, 0), (0, pad)))
    ch = jnp.transpose(boxes_p, (0, 2, 1))
    chg = jnp.concatenate([
        jnp.transpose(boxes_p.reshape(_B, _G, 128, 4), (0, 1, 3, 2)),
        jnp.transpose(scales_p.reshape(_B, _G, 128, _S), (0, 1, 3, 2)),
        ctx_p.reshape(_B, _G, 1, 128),
        scores_p.reshape(_B, _G, 1, 128),
    ], axis=2).reshape(_B, _G, _C * 128)
    tsc = target_scales.astype(jnp.int32)[:, :, None]
    tctx = target_context[:, :, None]
    out = pl.pallas_call(
        _loss_kernel,
        grid=(_B,),
        in_specs=[
            pl.BlockSpec((1, 4, _PP), lambda b: (b, 0, 0)),
            pl.BlockSpec((1, _G, _C * 128), lambda b: (b, 0, 0)),
            pl.BlockSpec((1, _N, 4), lambda b: (b, 0, 0)),
            pl.BlockSpec((1, _N, 1), lambda b: (b, 0, 0)),
            pl.BlockSpec((1, _N, 1), lambda b: (b, 0, 0)),
        ],
        out_specs=pl.BlockSpec((1, 1, 128), lambda b: (b, 0, 0)),
        out_shape=jax.ShapeDtypeStruct((_B, 1, 128), jnp.float32),
        scratch_shapes=[pltpu.VMEM((6, _N, _PCL), jnp.float32)],
        compiler_params=pltpu.CompilerParams(
            dimension_semantics=("parallel",)),
    )(ch, chg, target_boxes, tsc, tctx)
    res = out[:, 0, :]
    box_loss = jnp.mean(res[:, 0])
    scale_loss = jnp.mean(res[:, 1])
    context_loss = jnp.mean(res[:, 2])
    conf_loss = jnp.mean(res[:, 3])
    total = box_loss + scale_loss + context_loss + conf_loss
    return {"loss": total, "box_loss": box_loss, "scale_loss": scale_loss,
            "context_loss": context_loss, "conf_loss": conf_loss}
